# baseline (speedup 1.0000x reference)
import functools

import jax
import jax.numpy as jnp
from jax.experimental import pallas as pl


def _body_a(x_ref, a_ref, idx_ref, vm_ref, *, R):
    B, ER = vm_ref.shape
    x = x_ref[...].astype(jnp.bfloat16)
    v = jax.lax.dot_general(
        x, a_ref[...],
        dimension_numbers=(((1,), (1,)), ((), ())),
        preferred_element_type=jnp.float32,
    )
    idx = idx_ref[0]
    lane_adapter = jax.lax.broadcasted_iota(jnp.int32, (B, ER), 1) // R
    vm_ref[...] = jnp.where(lane_adapter == idx, v, 0.0).astype(jnp.bfloat16)


def _body_b(vm_ref, res_ref, b_ref, out_ref):
    y = jax.lax.dot_general(
        vm_ref[...], b_ref[...],
        dimension_numbers=(((1,), (0,)), ((), ())),
        preferred_element_type=jnp.float32,
    )
    out_ref[...] = res_ref[...] + y


def kernel(result, input, lora_a, lora_b, adapter_indices):
    N, D = input.shape
    E, _L, R, _D = lora_a.shape
    ER = E * R
    B = 512 if N % 512 == 0 else 256
    NB = N // B

    a_bf = lora_a[:, 0].reshape(ER, D).astype(jnp.bfloat16)
    b_bf = lora_b[:, 0].reshape(ER, D).astype(jnp.bfloat16)
    idx3 = adapter_indices.astype(jnp.int32).reshape(NB, B, 1)

    vm = pl.pallas_call(
        functools.partial(_body_a, R=R),
        grid=(NB,),
        in_specs=[
            pl.BlockSpec((B, D), lambda i: (i, 0)),
            pl.BlockSpec((ER, D), lambda i: (0, 0)),
            pl.BlockSpec((1, B, 1), lambda i: (i, 0, 0)),
        ],
        out_specs=pl.BlockSpec((B, ER), lambda i: (i, 0)),
        out_shape=jax.ShapeDtypeStruct((N, ER), jnp.bfloat16),
    )(input, a_bf, idx3)

    out = pl.pallas_call(
        _body_b,
        grid=(NB,),
        in_specs=[
            pl.BlockSpec((B, ER), lambda i: (i, 0)),
            pl.BlockSpec((B, D), lambda i: (i, 0)),
            pl.BlockSpec((ER, D), lambda i: (0, 0)),
        ],
        out_specs=pl.BlockSpec((B, D), lambda i: (i, 0)),
        out_shape=jax.ShapeDtypeStruct((N, D), jnp.float32),
    )(vm, result, b_bf)
    return out
